# Initial kernel scaffold; baseline (speedup 1.0000x reference)
import jax
import jax.numpy as jnp
from jax.experimental import pallas as pl


def kernel(xyz, norm, points, W0, b0, g0, be0, W1, b1, g1, be1, W2, b2, g2, be2, W3, b3, g3, be3):
    raise NotImplementedError("write your pallas kernel here")


# speedup vs baseline: 1.0844x; 1.0844x over previous
import jax
import jax.numpy as jnp
from jax.experimental import pallas as pl

_K = 32
_QT = 256
_LT = 2048


def _knn_body(xq_ref, xp_ref, out_ref):
    xq = xq_ref[0]
    xp = xp_ref[0]
    d = (xq[:, 0:1] - xp[:, 0][None, :]) ** 2
    d = d + (xq[:, 1:2] - xp[:, 1][None, :]) ** 2
    d = d + (xq[:, 2:3] - xp[:, 2][None, :]) ** 2
    iota = jax.lax.broadcasted_iota(jnp.int32, d.shape, 1)
    idxs = []
    for _ in range(_K):
        am = jnp.argmin(d, axis=1).astype(jnp.int32)
        idxs.append(am)
        d = jnp.where(iota == am[:, None], jnp.inf, d)
    out_ref[0] = jnp.stack(idxs, axis=1)


def _knn(xyz):
    B, N, _ = xyz.shape
    grid = (B, N // _QT)
    return pl.pallas_call(
        _knn_body,
        grid=grid,
        in_specs=[
            pl.BlockSpec((1, _QT, 3), lambda b, q: (b, q, 0)),
            pl.BlockSpec((1, N, 3), lambda b, q: (b, 0, 0)),
        ],
        out_specs=pl.BlockSpec((1, _QT, _K), lambda b, q: (b, q, 0)),
        out_shape=jax.ShapeDtypeStruct((B, N, _K), jnp.int32),
    )(xyz, xyz)


def _mm_body(w_ref, x_ref, o_ref):
    o_ref[0] = jnp.dot(w_ref[...], x_ref[0],
                       preferred_element_type=jnp.float32)


def _conv_mm(x, W):
    B, C, H, Wd = x.shape
    O = W.shape[0]
    L = H * Wd
    xf = x.reshape(B, C, L)
    lt = min(_LT, L)
    grid = (B, L // lt)
    y = pl.pallas_call(
        _mm_body,
        grid=grid,
        in_specs=[
            pl.BlockSpec((O, C), lambda b, l: (0, 0)),
            pl.BlockSpec((1, C, lt), lambda b, l: (b, 0, l)),
        ],
        out_specs=pl.BlockSpec((1, O, lt), lambda b, l: (b, 0, l)),
        out_shape=jax.ShapeDtypeStruct((B, O, L), jnp.float32),
    )(W, xf)
    return y.reshape(B, O, H, Wd)


def _safe_norm_unit(v):
    s = jnp.sum(v * v, axis=-1, keepdims=True)
    pos = s > 0
    l = jnp.sqrt(jnp.where(pos, s, 1.0))
    return jnp.where(pos, l, 0.0), jnp.where(pos, v / l, 0.0)


def _index_points(points, idx):
    B = points.shape[0]
    batch = jnp.arange(B).reshape((B,) + (1,) * (idx.ndim - 1))
    return points[batch, idx]


def _order_index(xyz, new_xyz, new_norm, idx):
    B, S, C = new_xyz.shape
    ns = idx.shape[2]
    grouped = _index_points(xyz, idx)
    local = grouped - new_xyz[:, :, None, :]
    dist_plane = jnp.matmul(local, new_norm)
    proj = local - dist_plane * jnp.reshape(new_norm, (B, S, 1, C))
    proj_len, proj_unit = _safe_norm_unit(proj)
    lm = jnp.argmax(proj_len, axis=2)
    vec_ref = jnp.take_along_axis(
        proj_unit, jnp.broadcast_to(lm[:, :, :, None], (B, S, 1, C)), axis=2)
    dots = jnp.matmul(proj_unit, jnp.reshape(vec_ref, (B, S, C, 1)))
    sgn = jnp.cross(proj_unit,
                    jnp.broadcast_to(jnp.reshape(vec_ref, (B, S, 1, C)),
                                     (B, S, ns, C)))
    sgn = jnp.sign(jnp.matmul(sgn, new_norm))
    sgn = sgn.at[:, :, 0, 0].set(1.0)
    dots = sgn * dots - (1 - sgn)
    order = jnp.argsort(-dots[..., 0], axis=2)
    dots_sorted = jnp.take_along_axis(dots, order[..., None], axis=2)
    idx_ordered = jnp.take_along_axis(idx, order, axis=2)
    return dots_sorted, idx_ordered


def _ri_features(xyz, norm, new_xyz, new_norm, idx):
    nn4 = new_norm[..., None]
    dots_sorted, idx_ordered = _order_index(xyz, new_xyz, nn4, idx)
    eps = 1e-07
    grouped = _index_points(xyz, idx_ordered)
    local = grouped - new_xyz[:, :, None, :]
    g_len, g_unit = _safe_norm_unit(local)
    g_norm = _index_points(norm, idx_ordered)
    a0 = jnp.matmul(g_unit, nn4)
    a1 = jnp.sum(g_unit * g_norm, axis=-1, keepdims=True)
    an = jnp.arccos(jnp.clip(jnp.matmul(g_norm, nn4), -1 + eps, 1 - eps))
    an = jnp.where(a0 < a1, 1.0, -1.0) * an
    inner = local - jnp.roll(local, 1, axis=2)
    _, i_unit = _safe_norm_unit(inner)
    ia0 = jnp.sum(i_unit * g_norm, axis=-1, keepdims=True)
    ia1 = jnp.sum(i_unit * jnp.roll(g_norm, 1, axis=2), axis=-1, keepdims=True)
    ia2 = jnp.arccos(jnp.clip(
        jnp.sum(g_norm * jnp.roll(g_norm, 1, axis=2), axis=-1, keepdims=True),
        -1 + eps, 1 - eps))
    ia2 = jnp.where(ia0 < ia1, 1.0, -1.0) * ia2
    pf = dots_sorted - jnp.roll(dots_sorted, 1, axis=2)
    pf = pf.at[:, :, 0, 0].set(-3 - dots_sorted[:, :, -1, 0])
    ri = jnp.concatenate([g_len, pf, a0, a1, an, ia0, ia1, ia2], axis=-1)
    return ri, idx_ordered


def _conv_bn_relu(x, W, b, g, beta):
    y = _conv_mm(x, W) + b[None, :, None, None]
    m = jnp.mean(y, axis=(0, 2, 3), keepdims=True)
    v = jnp.mean((y - m) ** 2, axis=(0, 2, 3), keepdims=True)
    y = (y - m) / jnp.sqrt(v + 1e-05)
    return jax.nn.relu(y * g[None, :, None, None] + beta[None, :, None, None])


def kernel(xyz, norm, points, W0, b0, g0, be0, W1, b1, g1, be1,
           W2, b2, g2, be2, W3, b3, g3, be3):
    idx = _knn(xyz)
    ri, idx_ordered = _ri_features(xyz, norm, xyz, norm, idx)
    x = jnp.transpose(ri, (0, 3, 2, 1))
    x = _conv_bn_relu(x, W0, b0, g0, be0)
    x = _conv_bn_relu(x, W1, b1, g1, be1)
    gp = jnp.transpose(_index_points(points, idx_ordered), (0, 3, 2, 1))
    x = jnp.concatenate([x, gp], axis=1)
    x = _conv_bn_relu(x, W2, b2, g2, be2)
    x = _conv_bn_relu(x, W3, b3, g3, be3)
    return jnp.transpose(jnp.mean(x, axis=2), (0, 2, 1)), idx


# speedup vs baseline: 1.0851x; 1.0007x over previous
import jax
import jax.numpy as jnp
from jax.experimental import pallas as pl

_K = 32
_QT = 512
_LT = 2048


def _knn_body(xq_ref, xp_ref, out_ref):
    xq = xq_ref[0]
    xp = xp_ref[0]
    d = (xq[:, 0:1] - xp[:, 0][None, :]) ** 2
    d = d + (xq[:, 1:2] - xp[:, 1][None, :]) ** 2
    d = d + (xq[:, 2:3] - xp[:, 2][None, :]) ** 2
    iota = jax.lax.broadcasted_iota(jnp.int32, d.shape, 1)
    idxs = []
    for _ in range(_K):
        am = jnp.argmin(d, axis=1).astype(jnp.int32)
        idxs.append(am)
        d = jnp.where(iota == am[:, None], jnp.inf, d)
    out_ref[0] = jnp.stack(idxs, axis=1)


def _knn(xyz):
    B, N, _ = xyz.shape
    grid = (B, N // _QT)
    return pl.pallas_call(
        _knn_body,
        grid=grid,
        in_specs=[
            pl.BlockSpec((1, _QT, 3), lambda b, q: (b, q, 0)),
            pl.BlockSpec((1, N, 3), lambda b, q: (b, 0, 0)),
        ],
        out_specs=pl.BlockSpec((1, _QT, _K), lambda b, q: (b, q, 0)),
        out_shape=jax.ShapeDtypeStruct((B, N, _K), jnp.int32),
    )(xyz, xyz)


def _mm_body(w_ref, x_ref, o_ref):
    o_ref[0] = jnp.dot(w_ref[...], x_ref[0],
                       preferred_element_type=jnp.float32)


def _conv_mm(x, W):
    B, C, H, Wd = x.shape
    O = W.shape[0]
    L = H * Wd
    xf = x.reshape(B, C, L)
    lt = min(_LT, L)
    grid = (B, L // lt)
    y = pl.pallas_call(
        _mm_body,
        grid=grid,
        in_specs=[
            pl.BlockSpec((O, C), lambda b, l: (0, 0)),
            pl.BlockSpec((1, C, lt), lambda b, l: (b, 0, l)),
        ],
        out_specs=pl.BlockSpec((1, O, lt), lambda b, l: (b, 0, l)),
        out_shape=jax.ShapeDtypeStruct((B, O, L), jnp.float32),
    )(W, xf)
    return y.reshape(B, O, H, Wd)


def _safe_norm_unit(v):
    s = jnp.sum(v * v, axis=-1, keepdims=True)
    pos = s > 0
    l = jnp.sqrt(jnp.where(pos, s, 1.0))
    return jnp.where(pos, l, 0.0), jnp.where(pos, v / l, 0.0)


def _index_points(points, idx):
    B = points.shape[0]
    batch = jnp.arange(B).reshape((B,) + (1,) * (idx.ndim - 1))
    return points[batch, idx]


def _order_index(xyz, new_xyz, new_norm, idx):
    B, S, C = new_xyz.shape
    ns = idx.shape[2]
    grouped = _index_points(xyz, idx)
    local = grouped - new_xyz[:, :, None, :]
    dist_plane = jnp.matmul(local, new_norm)
    proj = local - dist_plane * jnp.reshape(new_norm, (B, S, 1, C))
    proj_len, proj_unit = _safe_norm_unit(proj)
    lm = jnp.argmax(proj_len, axis=2)
    vec_ref = jnp.take_along_axis(
        proj_unit, jnp.broadcast_to(lm[:, :, :, None], (B, S, 1, C)), axis=2)
    dots = jnp.matmul(proj_unit, jnp.reshape(vec_ref, (B, S, C, 1)))
    sgn = jnp.cross(proj_unit,
                    jnp.broadcast_to(jnp.reshape(vec_ref, (B, S, 1, C)),
                                     (B, S, ns, C)))
    sgn = jnp.sign(jnp.matmul(sgn, new_norm))
    sgn = sgn.at[:, :, 0, 0].set(1.0)
    dots = sgn * dots - (1 - sgn)
    order = jnp.argsort(-dots[..., 0], axis=2)
    dots_sorted = jnp.take_along_axis(dots, order[..., None], axis=2)
    idx_ordered = jnp.take_along_axis(idx, order, axis=2)
    return dots_sorted, idx_ordered


def _ri_features(xyz, norm, new_xyz, new_norm, idx):
    nn4 = new_norm[..., None]
    dots_sorted, idx_ordered = _order_index(xyz, new_xyz, nn4, idx)
    eps = 1e-07
    grouped = _index_points(xyz, idx_ordered)
    local = grouped - new_xyz[:, :, None, :]
    g_len, g_unit = _safe_norm_unit(local)
    g_norm = _index_points(norm, idx_ordered)
    a0 = jnp.matmul(g_unit, nn4)
    a1 = jnp.sum(g_unit * g_norm, axis=-1, keepdims=True)
    an = jnp.arccos(jnp.clip(jnp.matmul(g_norm, nn4), -1 + eps, 1 - eps))
    an = jnp.where(a0 < a1, 1.0, -1.0) * an
    inner = local - jnp.roll(local, 1, axis=2)
    _, i_unit = _safe_norm_unit(inner)
    ia0 = jnp.sum(i_unit * g_norm, axis=-1, keepdims=True)
    ia1 = jnp.sum(i_unit * jnp.roll(g_norm, 1, axis=2), axis=-1, keepdims=True)
    ia2 = jnp.arccos(jnp.clip(
        jnp.sum(g_norm * jnp.roll(g_norm, 1, axis=2), axis=-1, keepdims=True),
        -1 + eps, 1 - eps))
    ia2 = jnp.where(ia0 < ia1, 1.0, -1.0) * ia2
    pf = dots_sorted - jnp.roll(dots_sorted, 1, axis=2)
    pf = pf.at[:, :, 0, 0].set(-3 - dots_sorted[:, :, -1, 0])
    ri = jnp.concatenate([g_len, pf, a0, a1, an, ia0, ia1, ia2], axis=-1)
    return ri, idx_ordered


def _conv_bn_relu(x, W, b, g, beta):
    y = _conv_mm(x, W) + b[None, :, None, None]
    m = jnp.mean(y, axis=(0, 2, 3), keepdims=True)
    v = jnp.mean((y - m) ** 2, axis=(0, 2, 3), keepdims=True)
    y = (y - m) / jnp.sqrt(v + 1e-05)
    return jax.nn.relu(y * g[None, :, None, None] + beta[None, :, None, None])


def kernel(xyz, norm, points, W0, b0, g0, be0, W1, b1, g1, be1,
           W2, b2, g2, be2, W3, b3, g3, be3):
    idx = _knn(xyz)
    ri, idx_ordered = _ri_features(xyz, norm, xyz, norm, idx)
    x = jnp.transpose(ri, (0, 3, 2, 1))
    x = _conv_bn_relu(x, W0, b0, g0, be0)
    x = _conv_bn_relu(x, W1, b1, g1, be1)
    gp = jnp.transpose(_index_points(points, idx_ordered), (0, 3, 2, 1))
    x = jnp.concatenate([x, gp], axis=1)
    x = _conv_bn_relu(x, W2, b2, g2, be2)
    x = _conv_bn_relu(x, W3, b3, g3, be3)
    return jnp.transpose(jnp.mean(x, axis=2), (0, 2, 1)), idx
